# Initial kernel scaffold; baseline (speedup 1.0000x reference)
import jax
import jax.numpy as jnp
from jax.experimental import pallas as pl


def kernel(x, edge_index, W1, b1, W2, b2, CW1, Cb1, CW2, Cb2):
    raise NotImplementedError("write your pallas kernel here")


# speedup vs baseline: 52.8154x; 52.8154x over previous
import functools

import jax
import jax.numpy as jnp
from jax import lax
from jax.experimental import pallas as pl
from jax.experimental.pallas import tpu as pltpu
from jax.experimental.pallas import tpu_sc as plsc

_N = 100000
_E = 3200000
_D = 16
_LANE = 128
_KB = 8
_CHUNK = _LANE * _KB
_NC = 2
_NTILE = 16
_NW = _NC * _NTILE
_NCHUNK = 98
_EPT = _CHUNK * _NCHUNK
_EPAD = _EPT * _NW
_BLKS = _EPAD // _LANE
_BPW = _BLKS // _NW
_PADR = 2400
_NT = _N + _PADR
_RPT = _NT // _NTILE

_mesh = plsc.VectorSubcoreMesh(core_axis_name="c", subcore_axis_name="s")


def _deg_body(dst_hbm, out_hbm, didx, ones_v, zbuf, deg_acc, sem):
    cid = lax.axis_index("c")
    sid = lax.axis_index("s")
    wid = sid * _NC + cid
    tid = sid

    def fill(i, c):
        ones_v[pl.ds(i * 16, 16)] = jnp.full((16,), 1.0, jnp.float32)
        return c

    lax.fori_loop(0, _CHUNK // 16, fill, 0)

    def zfill(i, c):
        zbuf[pl.ds(i * 16, 16)] = jnp.zeros((16,), jnp.float32)
        return c

    lax.fori_loop(0, _RPT // 16, zfill, 0)
    pltpu.sync_copy(zbuf, deg_acc.at[pl.ds(tid * _RPT, _RPT)])
    plsc.subcore_barrier()

    base = wid * _BPW

    def chunk(c, carry):
        row0 = base + c * _KB
        pltpu.sync_copy(dst_hbm.at[pl.ds(row0, _KB)], didx)
        handles = [
            pltpu.async_copy(
                ones_v.at[pl.ds(j * _LANE, _LANE)],
                deg_acc.at[didx.at[j]],
                sem,
                add=True,
            )
            for j in range(_KB)
        ]
        for h in handles:
            h.wait()
        return carry

    lax.fori_loop(0, _NCHUNK, chunk, 0)
    plsc.subcore_barrier()
    pltpu.sync_copy(
        deg_acc.at[pl.ds(tid * _RPT, _RPT)],
        out_hbm.at[cid, pl.ds(tid * _RPT, _RPT)],
    )


_SC_PARAMS = pltpu.CompilerParams(use_tc_tiling_on_sc=False)

_deg = pl.kernel(
    _deg_body,
    out_type=jax.ShapeDtypeStruct((_NC, _NT), jnp.float32),
    mesh=_mesh,
    compiler_params=_SC_PARAMS,
    scratch_types=[
        pltpu.VMEM((_KB, _LANE), jnp.int32),
        pltpu.VMEM((_CHUNK,), jnp.float32),
        pltpu.VMEM((_RPT,), jnp.float32),
        pltpu.VMEM_SHARED((_NT,), jnp.float32),
        pltpu.SemaphoreType.DMA,
    ],
)


def _agg_body(g_hbm, src_hbm, dst_hbm, out_hbm, sidx, didx, rows, acc,
              sem_g, sem_s):
    cid = lax.axis_index("c")
    sid = lax.axis_index("s")
    wid = sid * _NC + cid
    tid = sid

    def zrow(i, c):
        rows[i, :] = jnp.zeros((16,), jnp.float32)
        return c

    lax.fori_loop(0, _CHUNK, zrow, 0)
    for k in range(_RPT // _CHUNK):
        pltpu.sync_copy(rows, acc.at[pl.ds(tid * _RPT + k * _CHUNK, _CHUNK)])
    _TAIL = _RPT % _CHUNK
    pltpu.sync_copy(
        rows.at[pl.ds(0, _TAIL)],
        acc.at[pl.ds(tid * _RPT + _RPT - _TAIL, _TAIL)],
    )
    plsc.subcore_barrier()

    base = wid * _BPW

    def chunk(c, carry):
        row0 = base + c * _KB
        pltpu.sync_copy(src_hbm.at[pl.ds(row0, _KB)], sidx)
        pltpu.sync_copy(dst_hbm.at[pl.ds(row0, _KB)], didx)
        gh = [
            pltpu.async_copy(
                g_hbm.at[sidx.at[j]],
                rows.at[pl.ds(j * _LANE, _LANE)],
                sem_g,
            )
            for j in range(_KB)
        ]
        for h in gh:
            h.wait()
        sh = [
            pltpu.async_copy(
                rows.at[pl.ds(j * _LANE, _LANE)],
                acc.at[didx.at[j]],
                sem_s,
                add=True,
            )
            for j in range(_KB)
        ]
        for h in sh:
            h.wait()
        return carry

    lax.fori_loop(0, _NCHUNK, chunk, 0)
    plsc.subcore_barrier()
    pltpu.sync_copy(
        acc.at[pl.ds(tid * _RPT, _RPT)],
        out_hbm.at[cid, pl.ds(tid * _RPT, _RPT)],
    )


_agg = pl.kernel(
    _agg_body,
    out_type=jax.ShapeDtypeStruct((_NC, _NT, _D), jnp.float32),
    mesh=_mesh,
    compiler_params=_SC_PARAMS,
    scratch_types=[
        pltpu.VMEM((_KB, _LANE), jnp.int32),
        pltpu.VMEM((_KB, _LANE), jnp.int32),
        pltpu.VMEM((_CHUNK, _D), jnp.float32),
        pltpu.VMEM_SHARED((_NT, _D), jnp.float32),
        pltpu.SemaphoreType.DMA,
        pltpu.SemaphoreType.DMA,
    ],
)


_R = 2000


def _tc1_body(x_ref, w1_ref, p0_ref, p1_ref, g_ref, dinv_ref):
    deg = p0_ref[...] + p1_ref[...] + 1.0
    dinv = lax.rsqrt(deg)
    h = jnp.dot(x_ref[...], w1_ref[...], preferred_element_type=jnp.float32)
    g_ref[...] = h * dinv
    dinv_ref[...] = dinv


def _tc2_body(a0_ref, a1_ref, g1_ref, dinv_ref, b1_ref, w2_ref, g2_ref):
    dinv = dinv_ref[...]
    z = (a0_ref[...] + a1_ref[...] + g1_ref[...]) * dinv + b1_ref[...]
    z = jnp.maximum(z, 0.0)
    g2_ref[...] = jnp.dot(z, w2_ref[...],
                          preferred_element_type=jnp.float32) * dinv


def _tc3_body(a0_ref, a1_ref, g2_ref, dinv_ref, b2_ref, cw1_ref, cb1_ref,
              cw2_ref, cb2_ref, o1_ref, o2_ref):
    z = (a0_ref[...] + a1_ref[...] + g2_ref[...]) * dinv_ref[...] + b2_ref[...]
    z = jnp.maximum(z, 0.0)
    o1_ref[...] = jnp.dot(z, cw1_ref[...],
                          preferred_element_type=jnp.float32) + cb1_ref[...]
    o2_ref[...] = jnp.dot(z, cw2_ref[...],
                          preferred_element_type=jnp.float32) + cb2_ref[...]


def _row_spec(w):
    return pl.BlockSpec((_R, w), lambda i: (i, 0))


def _full_spec(h, w):
    return pl.BlockSpec((h, w), lambda i: (0, 0))


_GRID = (_N // _R,)

_tc1 = pl.pallas_call(
    _tc1_body,
    grid=_GRID,
    in_specs=[_row_spec(6), _full_spec(6, _D), _row_spec(1), _row_spec(1)],
    out_specs=[_row_spec(_D), _row_spec(1)],
    out_shape=[
        jax.ShapeDtypeStruct((_N, _D), jnp.float32),
        jax.ShapeDtypeStruct((_N, 1), jnp.float32),
    ],
)

_tc2 = pl.pallas_call(
    _tc2_body,
    grid=_GRID,
    in_specs=[_row_spec(_D), _row_spec(_D), _row_spec(_D), _row_spec(1),
              _full_spec(1, _D), _full_spec(_D, _D)],
    out_specs=[_row_spec(_D)],
    out_shape=[jax.ShapeDtypeStruct((_N, _D), jnp.float32)],
)

_tc3 = pl.pallas_call(
    _tc3_body,
    grid=_GRID,
    in_specs=[_row_spec(_D), _row_spec(_D), _row_spec(_D), _row_spec(1),
              _full_spec(1, _D), _full_spec(_D, 13), _full_spec(1, 13),
              _full_spec(_D, 8), _full_spec(1, 8)],
    out_specs=[_row_spec(13), _row_spec(8)],
    out_shape=[
        jax.ShapeDtypeStruct((_N, 13), jnp.float32),
        jax.ShapeDtypeStruct((_N, 8), jnp.float32),
    ],
)


def kernel(x, edge_index, W1, b1, W2, b2, CW1, Cb1, CW2, Cb2):
    src = edge_index[0]
    dst = edge_index[1]
    pad_i = jnp.arange(_EPAD - _E, dtype=jnp.int32)
    srcp = jnp.concatenate([src, pad_i % _N])
    dstp = jnp.concatenate([dst, _N + pad_i % _PADR])
    src2d = srcp.reshape(_BLKS, _LANE)
    dst2d = dstp.reshape(_BLKS, _LANE)

    degp = _deg(dst2d)
    p0 = degp[0, :_N].reshape(_N, 1)
    p1 = degp[1, :_N].reshape(_N, 1)

    g1, dinv = _tc1(x, W1, p0, p1)
    acc1 = _agg(g1, src2d, dst2d)
    (g2,) = _tc2(acc1[0, :_N], acc1[1, :_N], g1, dinv,
                 b1.reshape(1, _D), W2)
    acc2 = _agg(g2, src2d, dst2d)
    out1, out2 = _tc3(acc2[0, :_N], acc2[1, :_N], g2, dinv,
                      b2.reshape(1, _D), CW1, Cb1.reshape(1, 13),
                      CW2, Cb2.reshape(1, 8))
    return (out1, out2)


# speedup vs baseline: 72.4610x; 1.3720x over previous
import jax
import jax.numpy as jnp
import numpy as np
from jax import lax
from jax.experimental import pallas as pl
from jax.experimental.pallas import tpu as pltpu
from jax.experimental.pallas import tpu_sc as plsc

_N = 100000
_E = 3200000
_D = 16
_LANE = 128
_KB = 8
_CHUNK = _LANE * _KB
_NC = 2
_NTILE = 16
_NW = _NC * _NTILE
_NCHUNK = 98
_EPT = _CHUNK * _NCHUNK
_EPAD = _EPT * _NW
_BLKS = _EPAD // _LANE
_BPW = _BLKS // _NW
_NP = 102400
_PADR = _NP - _N
_NT = _NP
_RPT = _NT // _NTILE

_FOLD = 8
_NF = _NP // _FOLD
_NDEG = _NP // _LANE

_mesh = plsc.VectorSubcoreMesh(core_axis_name="c", subcore_axis_name="s")


def _deg_body(dst_hbm, out_hbm, didx, ones_v, zbuf, deg_acc, sem):
    cid = lax.axis_index("c")
    sid = lax.axis_index("s")
    wid = sid * _NC + cid
    tid = sid

    def fill(i, c):
        ones_v[pl.ds(i * 16, 16)] = jnp.full((16,), 1.0, jnp.float32)
        return c

    lax.fori_loop(0, _CHUNK // 16, fill, 0)

    def zfill(i, c):
        zbuf[pl.ds(i * 16, 16)] = jnp.zeros((16,), jnp.float32)
        return c

    lax.fori_loop(0, _RPT // 16, zfill, 0)
    pltpu.sync_copy(zbuf, deg_acc.at[pl.ds(tid * _RPT, _RPT)])
    plsc.subcore_barrier()

    base = wid * _BPW

    def chunk(c, carry):
        row0 = base + c * _KB
        pltpu.sync_copy(dst_hbm.at[pl.ds(row0, _KB)], didx)
        handles = [
            pltpu.async_copy(
                ones_v.at[pl.ds(j * _LANE, _LANE)],
                deg_acc.at[didx.at[j]],
                sem,
                add=True,
            )
            for j in range(_KB)
        ]
        for h in handles:
            h.wait()
        return carry

    lax.fori_loop(0, _NCHUNK, chunk, 0)
    plsc.subcore_barrier()
    pltpu.sync_copy(
        deg_acc.at[pl.ds(tid * _RPT, _RPT)],
        out_hbm.at[cid, pl.ds(tid * _RPT, _RPT)],
    )


_SC_PARAMS = pltpu.CompilerParams(use_tc_tiling_on_sc=False)

_deg = pl.kernel(
    _deg_body,
    out_type=jax.ShapeDtypeStruct((_NC, _NT), jnp.float32),
    mesh=_mesh,
    compiler_params=_SC_PARAMS,
    scratch_types=[
        pltpu.VMEM((_KB, _LANE), jnp.int32),
        pltpu.VMEM((_CHUNK,), jnp.float32),
        pltpu.VMEM((_RPT,), jnp.float32),
        pltpu.VMEM_SHARED((_NT,), jnp.float32),
        pltpu.SemaphoreType.DMA,
    ],
)


def _agg_body(g_hbm, src_hbm, dst_hbm, out_hbm, sidx, didx, rows, acc,
              sem_g, sem_s):
    cid = lax.axis_index("c")
    sid = lax.axis_index("s")
    wid = sid * _NC + cid
    tid = sid

    def zrow(i, c):
        rows[i, :] = jnp.zeros((16,), jnp.float32)
        return c

    lax.fori_loop(0, _CHUNK, zrow, 0)
    for k in range(_RPT // _CHUNK):
        pltpu.sync_copy(rows, acc.at[pl.ds(tid * _RPT + k * _CHUNK, _CHUNK)])
    _TAIL = _RPT % _CHUNK
    pltpu.sync_copy(
        rows.at[pl.ds(0, _TAIL)],
        acc.at[pl.ds(tid * _RPT + _RPT - _TAIL, _TAIL)],
    )
    plsc.subcore_barrier()

    base = wid * _BPW

    def chunk(c, carry):
        row0 = base + c * _KB
        pltpu.sync_copy(src_hbm.at[pl.ds(row0, _KB)], sidx)
        pltpu.sync_copy(dst_hbm.at[pl.ds(row0, _KB)], didx)
        gh = [
            pltpu.async_copy(
                g_hbm.at[sidx.at[j]],
                rows.at[pl.ds(j * _LANE, _LANE)],
                sem_g,
            )
            for j in range(_KB)
        ]
        for h in gh:
            h.wait()
        sh = [
            pltpu.async_copy(
                rows.at[pl.ds(j * _LANE, _LANE)],
                acc.at[didx.at[j]],
                sem_s,
                add=True,
            )
            for j in range(_KB)
        ]
        for h in sh:
            h.wait()
        return carry

    lax.fori_loop(0, _NCHUNK, chunk, 0)
    plsc.subcore_barrier()
    pltpu.sync_copy(
        acc.at[pl.ds(tid * _RPT, _RPT)],
        out_hbm.at[cid, pl.ds(tid * _RPT, _RPT)],
    )


_agg = pl.kernel(
    _agg_body,
    out_type=jax.ShapeDtypeStruct((_NC, _NT, _D), jnp.float32),
    mesh=_mesh,
    compiler_params=_SC_PARAMS,
    scratch_types=[
        pltpu.VMEM((_KB, _LANE), jnp.int32),
        pltpu.VMEM((_KB, _LANE), jnp.int32),
        pltpu.VMEM((_CHUNK, _D), jnp.float32),
        pltpu.VMEM_SHARED((_NT, _D), jnp.float32),
        pltpu.SemaphoreType.DMA,
        pltpu.SemaphoreType.DMA,
    ],
)


_RB = 256
_DB = _RB // _NTILE
_GRID = (_NF // _RB,)


def _expand_dinv(p0, p1):
    dinvp = lax.rsqrt(p0 + p1 + 1.0)
    rep = jnp.broadcast_to(dinvp[:, None, :], (_DB, 16, _LANE))
    rep = rep.reshape(_RB, _LANE)
    rr = lax.broadcasted_iota(jnp.int32, (_RB, _LANE), 0)
    cc = lax.broadcasted_iota(jnp.int32, (_RB, _LANE), 1)
    lmask = (cc // _FOLD == rr % 16).astype(jnp.float32)
    x1 = rep * lmask
    rc = lax.broadcasted_iota(jnp.int32, (_LANE, _LANE), 0)
    rl = lax.broadcasted_iota(jnp.int32, (_LANE, _LANE), 1)
    rmat = (rl // _D == rc % _FOLD).astype(jnp.float32)
    return jnp.dot(x1, rmat, preferred_element_type=jnp.float32,
                   precision=lax.Precision.HIGHEST)


def _tc1_body(x_ref, w1_ref, p0_ref, p1_ref, g_ref, dinv_ref):
    wide = _expand_dinv(p0_ref[0], p1_ref[0])
    h = jnp.dot(x_ref[...], w1_ref[...], preferred_element_type=jnp.float32)
    g_ref[...] = h * wide
    dinv_ref[...] = wide


def _tc2_body(a0_ref, a1_ref, g1_ref, dinv_ref, b1_ref, w2_ref, g2_ref):
    dinv = dinv_ref[...]
    z = (a0_ref[0] + a1_ref[0] + g1_ref[...]) * dinv + b1_ref[...]
    z = jnp.maximum(z, 0.0)
    g2_ref[...] = jnp.dot(z, w2_ref[...],
                          preferred_element_type=jnp.float32) * dinv


def _tc3_body(a0_ref, a1_ref, g2_ref, dinv_ref, b2_ref, cw1_ref, cb1_ref,
              cw2_ref, cb2_ref, o1_ref, o2_ref):
    z = (a0_ref[0] + a1_ref[0] + g2_ref[...]) * dinv_ref[...] + b2_ref[...]
    z = jnp.maximum(z, 0.0)
    o1_ref[...] = jnp.dot(z, cw1_ref[...],
                          preferred_element_type=jnp.float32) + cb1_ref[...]
    o2_ref[...] = jnp.dot(z, cw2_ref[...],
                          preferred_element_type=jnp.float32) + cb2_ref[...]


def _row_spec(w):
    return pl.BlockSpec((_RB, w), lambda i: (i, 0))


def _full_spec(h, w):
    return pl.BlockSpec((h, w), lambda i: (0, 0))


def _part_spec(p, h, w):
    return pl.BlockSpec((1, h, w), lambda i, _p=p: (_p, i, 0))


_tc1 = pl.pallas_call(
    _tc1_body,
    grid=_GRID,
    in_specs=[_row_spec(48), _full_spec(48, _LANE),
              _part_spec(0, _DB, _LANE), _part_spec(1, _DB, _LANE)],
    out_specs=[_row_spec(_LANE), _row_spec(_LANE)],
    out_shape=[
        jax.ShapeDtypeStruct((_NF, _LANE), jnp.float32),
        jax.ShapeDtypeStruct((_NF, _LANE), jnp.float32),
    ],
)

_tc2 = pl.pallas_call(
    _tc2_body,
    grid=_GRID,
    in_specs=[_part_spec(0, _RB, _LANE), _part_spec(1, _RB, _LANE),
              _row_spec(_LANE), _row_spec(_LANE),
              _full_spec(1, _LANE), _full_spec(_LANE, _LANE)],
    out_specs=[_row_spec(_LANE)],
    out_shape=[jax.ShapeDtypeStruct((_NF, _LANE), jnp.float32)],
)

_tc3 = pl.pallas_call(
    _tc3_body,
    grid=_GRID,
    in_specs=[_part_spec(0, _RB, _LANE), _part_spec(1, _RB, _LANE),
              _row_spec(_LANE), _row_spec(_LANE),
              _full_spec(1, _LANE), _full_spec(_LANE, 104),
              _full_spec(1, 104), _full_spec(_LANE, 64), _full_spec(1, 64)],
    out_specs=[_row_spec(104), _row_spec(64)],
    out_shape=[
        jax.ShapeDtypeStruct((_NF, 104), jnp.float32),
        jax.ShapeDtypeStruct((_NF, 64), jnp.float32),
    ],
)

_EYE8 = np.eye(_FOLD, dtype=np.float32)


def kernel(x, edge_index, W1, b1, W2, b2, CW1, Cb1, CW2, Cb2):
    src = edge_index[0]
    dst = edge_index[1]
    pad_i = jnp.arange(_EPAD - _E, dtype=jnp.int32)
    srcp = jnp.concatenate([src, pad_i % _N])
    dstp = jnp.concatenate([dst, _N + pad_i % _PADR])
    src2d = srcp.reshape(_BLKS, _LANE)
    dst2d = dstp.reshape(_BLKS, _LANE)

    xp = jnp.concatenate(
        [x, jnp.zeros((_NP - _N, x.shape[1]), jnp.float32)])
    xf = xp.reshape(_NF, _FOLD * 6)

    w1big = jnp.kron(_EYE8, W1)
    w2big = jnp.kron(_EYE8, W2)
    cw1big = jnp.kron(_EYE8, CW1)
    cw2big = jnp.kron(_EYE8, CW2)
    b1t = jnp.tile(b1, _FOLD).reshape(1, _LANE)
    b2t = jnp.tile(b2, _FOLD).reshape(1, _LANE)
    cb1t = jnp.tile(Cb1, _FOLD).reshape(1, 104)
    cb2t = jnp.tile(Cb2, _FOLD).reshape(1, 64)

    degp = _deg(dst2d)
    dg = degp.reshape(_NC, _NDEG, _LANE)

    g1f, dinvw = _tc1(xf, w1big, dg, dg)
    acc1 = _agg(g1f.reshape(_NT, _D), src2d, dst2d)
    (g2f,) = _tc2(acc1.reshape(_NC, _NF, _LANE),
                  acc1.reshape(_NC, _NF, _LANE),
                  g1f, dinvw, b1t, w2big)
    acc2 = _agg(g2f.reshape(_NT, _D), src2d, dst2d)
    o1f, o2f = _tc3(acc2.reshape(_NC, _NF, _LANE),
                    acc2.reshape(_NC, _NF, _LANE),
                    g2f, dinvw, b2t, cw1big, cb1t, cw2big, cb2t)
    out1 = o1f.reshape(_NP, 13)[:_N]
    out2 = o2f.reshape(_NP, 8)[:_N]
    return (out1, out2)


# speedup vs baseline: 101.3111x; 1.3981x over previous
import jax
import jax.numpy as jnp
import numpy as np
from jax import lax
from jax.experimental import pallas as pl
from jax.experimental.pallas import tpu as pltpu
from jax.experimental.pallas import tpu_sc as plsc

_N = 100000
_E = 3200000
_D = 16
_LANE = 128
_KB = 4
_CHUNK = _LANE * _KB
_DKB = 8
_DCHUNK = _LANE * _DKB
_NC = 2
_NTILE = 16
_NW = _NC * _NTILE
_NCHUNK = 196
_DNCHUNK = 98
_EPT = _CHUNK * _NCHUNK
_EPAD = _EPT * _NW
_BLKS = _EPAD // _LANE
_BPW = _BLKS // _NW
_NP = 102400
_PADR = _NP - _N
_NT = _NP
_RPT = _NT // _NTILE

_FOLD = 8
_NF = _NP // _FOLD
_NDEG = _NP // _LANE

_mesh = plsc.VectorSubcoreMesh(core_axis_name="c", subcore_axis_name="s")


def _deg_body(dst_hbm, out_hbm, didx, ones_v, zbuf, deg_acc, sem, sem_i):
    cid = lax.axis_index("c")
    sid = lax.axis_index("s")
    wid = sid * _NC + cid
    tid = sid

    def fill(i, c):
        ones_v[pl.ds(i * 16, 16)] = jnp.full((16,), 1.0, jnp.float32)
        return c

    lax.fori_loop(0, _DCHUNK // 16, fill, 0)

    def zfill(i, c):
        zbuf[pl.ds(i * 16, 16)] = jnp.zeros((16,), jnp.float32)
        return c

    lax.fori_loop(0, _RPT // 16, zfill, 0)
    pltpu.sync_copy(zbuf, deg_acc.at[pl.ds(tid * _RPT, _RPT)])
    plsc.subcore_barrier()

    base = wid * _BPW

    def load_idx(slot, c):
        pltpu.async_copy(
            dst_hbm.at[pl.ds(base + c * _DKB, _DKB)], didx.at[slot], sem_i)

    def wait_idx(slot):
        pltpu.make_async_copy(
            dst_hbm.at[pl.ds(base, _DKB)], didx.at[slot], sem_i).wait()

    load_idx(0, 0)

    def chunk(c, carry):
        b = lax.rem(c, 2)

        @pl.when(c + 1 < _DNCHUNK)
        def _prefetch():
            load_idx(1 - b, c + 1)

        wait_idx(b)
        handles = [
            pltpu.async_copy(
                ones_v.at[pl.ds(j * _LANE, _LANE)],
                deg_acc.at[didx.at[b, j]],
                sem,
                add=True,
            )
            for j in range(_DKB)
        ]
        for h in handles:
            h.wait()
        return carry

    lax.fori_loop(0, _DNCHUNK, chunk, 0)
    plsc.subcore_barrier()
    pltpu.sync_copy(
        deg_acc.at[pl.ds(tid * _RPT, _RPT)],
        out_hbm.at[cid, pl.ds(tid * _RPT, _RPT)],
    )


_SC_PARAMS = pltpu.CompilerParams(use_tc_tiling_on_sc=False)

_deg = pl.kernel(
    _deg_body,
    out_type=jax.ShapeDtypeStruct((_NC, _NT), jnp.float32),
    mesh=_mesh,
    compiler_params=_SC_PARAMS,
    scratch_types=[
        pltpu.VMEM((2, _DKB, _LANE), jnp.int32),
        pltpu.VMEM((_DCHUNK,), jnp.float32),
        pltpu.VMEM((_RPT,), jnp.float32),
        pltpu.VMEM_SHARED((_NT,), jnp.float32),
        pltpu.SemaphoreType.DMA,
        pltpu.SemaphoreType.DMA,
    ],
)


def _agg_body(g_hbm, src_hbm, dst_hbm, out_hbm, sidx, didx, rows, acc,
              sem_g, sem_s, sem_i):
    cid = lax.axis_index("c")
    sid = lax.axis_index("s")
    wid = sid * _NC + cid
    tid = sid

    def zrow(i, c):
        rows[0, i, :] = jnp.zeros((16,), jnp.float32)
        return c

    lax.fori_loop(0, _CHUNK, zrow, 0)
    for k in range(_RPT // _CHUNK):
        pltpu.sync_copy(rows.at[0],
                        acc.at[pl.ds(tid * _RPT + k * _CHUNK, _CHUNK)])
    _TAIL = _RPT % _CHUNK
    pltpu.sync_copy(
        rows.at[0, pl.ds(0, _TAIL)],
        acc.at[pl.ds(tid * _RPT + _RPT - _TAIL, _TAIL)],
    )
    plsc.subcore_barrier()

    base = wid * _BPW

    def load_idx(slot, c):
        row0 = base + c * _KB
        pltpu.async_copy(src_hbm.at[pl.ds(row0, _KB)], sidx.at[slot], sem_i)
        pltpu.async_copy(dst_hbm.at[pl.ds(row0, _KB)], didx.at[slot], sem_i)

    def wait_idx(slot):
        pltpu.make_async_copy(
            src_hbm.at[pl.ds(base, _KB)], sidx.at[slot], sem_i).wait()
        pltpu.make_async_copy(
            dst_hbm.at[pl.ds(base, _KB)], didx.at[slot], sem_i).wait()

    def fire_gather(buf, slot):
        for j in range(_KB):
            pltpu.async_copy(
                g_hbm.at[sidx.at[slot, j]],
                rows.at[buf, pl.ds(j * _LANE, _LANE)],
                sem_g,
            )

    def wait_gather(buf):
        for j in range(_KB):
            pltpu.make_async_copy(
                g_hbm.at[sidx.at[0, j]],
                rows.at[buf, pl.ds(j * _LANE, _LANE)],
                sem_g,
            ).wait()

    load_idx(0, 0)
    load_idx(1, 1)
    wait_idx(0)
    fire_gather(0, 0)
    wait_idx(1)
    fire_gather(1, 1)
    load_idx(2, 2)
    load_idx(3, 3)

    def chunk(c, carry):
        b = lax.rem(c, 2)
        q = lax.rem(c, 4)
        wait_gather(b)
        sh = [
            pltpu.async_copy(
                rows.at[b, pl.ds(j * _LANE, _LANE)],
                acc.at[didx.at[q, j]],
                sem_s,
                add=True,
            )
            for j in range(_KB)
        ]
        for h in sh:
            h.wait()

        @pl.when(c + 4 < _NCHUNK)
        def _prefetch():
            load_idx(q, c + 4)

        @pl.when(c + 2 < _NCHUNK)
        def _next_gather():
            qn = lax.rem(c + 2, 4)
            wait_idx(qn)
            fire_gather(b, qn)

        return carry

    lax.fori_loop(0, _NCHUNK, chunk, 0)
    plsc.subcore_barrier()
    pltpu.sync_copy(
        acc.at[pl.ds(tid * _RPT, _RPT)],
        out_hbm.at[cid, pl.ds(tid * _RPT, _RPT)],
    )


_agg = pl.kernel(
    _agg_body,
    out_type=jax.ShapeDtypeStruct((_NC, _NT, _D), jnp.float32),
    mesh=_mesh,
    compiler_params=_SC_PARAMS,
    scratch_types=[
        pltpu.VMEM((4, _KB, _LANE), jnp.int32),
        pltpu.VMEM((4, _KB, _LANE), jnp.int32),
        pltpu.VMEM((2, _CHUNK, _D), jnp.float32),
        pltpu.VMEM_SHARED((_NT, _D), jnp.float32),
        pltpu.SemaphoreType.DMA,
        pltpu.SemaphoreType.DMA,
        pltpu.SemaphoreType.DMA,
    ],
)


_RB = 256
_DB = _RB // _NTILE
_GRID = (_NF // _RB,)


def _expand_dinv(p0, p1):
    dinvp = lax.rsqrt(p0 + p1 + 1.0)
    rep = jnp.broadcast_to(dinvp[:, None, :], (_DB, 16, _LANE))
    rep = rep.reshape(_RB, _LANE)
    rr = lax.broadcasted_iota(jnp.int32, (_RB, _LANE), 0)
    cc = lax.broadcasted_iota(jnp.int32, (_RB, _LANE), 1)
    lmask = (cc // _FOLD == rr % 16).astype(jnp.float32)
    x1 = rep * lmask
    rc = lax.broadcasted_iota(jnp.int32, (_LANE, _LANE), 0)
    rl = lax.broadcasted_iota(jnp.int32, (_LANE, _LANE), 1)
    rmat = (rl // _D == rc % _FOLD).astype(jnp.float32)
    return jnp.dot(x1, rmat, preferred_element_type=jnp.float32,
                   precision=lax.Precision.HIGHEST)


def _tc1_body(x_ref, w1_ref, p0_ref, p1_ref, g_ref, dinv_ref):
    wide = _expand_dinv(p0_ref[0], p1_ref[0])
    h = jnp.dot(x_ref[...], w1_ref[...], preferred_element_type=jnp.float32)
    g_ref[...] = h * wide
    dinv_ref[...] = wide


def _tc2_body(a0_ref, a1_ref, g1_ref, dinv_ref, b1_ref, w2_ref, g2_ref):
    dinv = dinv_ref[...]
    z = (a0_ref[0] + a1_ref[0] + g1_ref[...]) * dinv + b1_ref[...]
    z = jnp.maximum(z, 0.0)
    g2_ref[...] = jnp.dot(z, w2_ref[...],
                          preferred_element_type=jnp.float32) * dinv


def _tc3_body(a0_ref, a1_ref, g2_ref, dinv_ref, b2_ref, cw1_ref, cb1_ref,
              cw2_ref, cb2_ref, o1_ref, o2_ref):
    z = (a0_ref[0] + a1_ref[0] + g2_ref[...]) * dinv_ref[...] + b2_ref[...]
    z = jnp.maximum(z, 0.0)
    o1_ref[...] = jnp.dot(z, cw1_ref[...],
                          preferred_element_type=jnp.float32) + cb1_ref[...]
    o2_ref[...] = jnp.dot(z, cw2_ref[...],
                          preferred_element_type=jnp.float32) + cb2_ref[...]


def _row_spec(w):
    return pl.BlockSpec((_RB, w), lambda i: (i, 0))


def _full_spec(h, w):
    return pl.BlockSpec((h, w), lambda i: (0, 0))


def _part_spec(p, h, w):
    return pl.BlockSpec((1, h, w), lambda i, _p=p: (_p, i, 0))


_tc1 = pl.pallas_call(
    _tc1_body,
    grid=_GRID,
    in_specs=[_row_spec(48), _full_spec(48, _LANE),
              _part_spec(0, _DB, _LANE), _part_spec(1, _DB, _LANE)],
    out_specs=[_row_spec(_LANE), _row_spec(_LANE)],
    out_shape=[
        jax.ShapeDtypeStruct((_NF, _LANE), jnp.float32),
        jax.ShapeDtypeStruct((_NF, _LANE), jnp.float32),
    ],
)

_tc2 = pl.pallas_call(
    _tc2_body,
    grid=_GRID,
    in_specs=[_part_spec(0, _RB, _LANE), _part_spec(1, _RB, _LANE),
              _row_spec(_LANE), _row_spec(_LANE),
              _full_spec(1, _LANE), _full_spec(_LANE, _LANE)],
    out_specs=[_row_spec(_LANE)],
    out_shape=[jax.ShapeDtypeStruct((_NF, _LANE), jnp.float32)],
)

_tc3 = pl.pallas_call(
    _tc3_body,
    grid=_GRID,
    in_specs=[_part_spec(0, _RB, _LANE), _part_spec(1, _RB, _LANE),
              _row_spec(_LANE), _row_spec(_LANE),
              _full_spec(1, _LANE), _full_spec(_LANE, _LANE),
              _full_spec(1, _LANE), _full_spec(_LANE, _LANE),
              _full_spec(1, _LANE)],
    out_specs=[_row_spec(_LANE), _row_spec(_LANE)],
    out_shape=[
        jax.ShapeDtypeStruct((_NF, _LANE), jnp.float32),
        jax.ShapeDtypeStruct((_NF, _LANE), jnp.float32),
    ],
)

_EYE8 = np.eye(_FOLD, dtype=np.float32)


def kernel(x, edge_index, W1, b1, W2, b2, CW1, Cb1, CW2, Cb2):
    src = edge_index[0]
    dst = edge_index[1]
    pad_i = jnp.arange(_EPAD - _E, dtype=jnp.int32)
    srcp = jnp.concatenate([src, pad_i % _N])
    dstp = jnp.concatenate([dst, _N + pad_i % _PADR])
    src2d = srcp.reshape(_BLKS, _LANE)
    dst2d = dstp.reshape(_BLKS, _LANE)

    xr = x.reshape(_N // _FOLD, _FOLD * 6)
    xf = jnp.concatenate(
        [xr, jnp.zeros((_NF - _N // _FOLD, _FOLD * 6), jnp.float32)])

    cw1p = jnp.pad(CW1, ((0, 0), (0, _D - 13)))
    cw2p = jnp.pad(CW2, ((0, 0), (0, _D - 8)))
    cb1p = jnp.pad(Cb1, (0, _D - 13))
    cb2p = jnp.pad(Cb2, (0, _D - 8))
    w1big = jnp.kron(_EYE8, W1)
    w2big = jnp.kron(_EYE8, W2)
    cw1big = jnp.kron(_EYE8, cw1p)
    cw2big = jnp.kron(_EYE8, cw2p)
    b1t = jnp.tile(b1, _FOLD).reshape(1, _LANE)
    b2t = jnp.tile(b2, _FOLD).reshape(1, _LANE)
    cb1t = jnp.tile(cb1p, _FOLD).reshape(1, _LANE)
    cb2t = jnp.tile(cb2p, _FOLD).reshape(1, _LANE)

    degp = _deg(dst2d)
    dg = degp.reshape(_NC, _NDEG, _LANE)

    g1f, dinvw = _tc1(xf, w1big, dg, dg)
    acc1 = _agg(g1f.reshape(_NT, _D), src2d, dst2d)
    (g2f,) = _tc2(acc1.reshape(_NC, _NF, _LANE),
                  acc1.reshape(_NC, _NF, _LANE),
                  g1f, dinvw, b1t, w2big)
    acc2 = _agg(g2f.reshape(_NT, _D), src2d, dst2d)
    o1f, o2f = _tc3(acc2.reshape(_NC, _NF, _LANE),
                    acc2.reshape(_NC, _NF, _LANE),
                    g2f, dinvw, b2t, cw1big, cb1t, cw2big, cb2t)
    out1 = o1f.reshape(_NP, _D)[:_N, :13]
    out2 = o2f.reshape(_NP, _D)[:_N, :8]
    return (out1, out2)


# speedup vs baseline: 109.0599x; 1.0765x over previous
import jax
import jax.numpy as jnp
import numpy as np
from jax import lax
from jax.experimental import pallas as pl
from jax.experimental.pallas import tpu as pltpu
from jax.experimental.pallas import tpu_sc as plsc

_N = 100000
_E = 3200000
_D = 16
_LANE = 128
_KB = 4
_CHUNK = _LANE * _KB
_DKB = 8
_DCHUNK = _LANE * _DKB
_NC = 2
_NTILE = 16
_NW = _NC * _NTILE
_NCHUNK = 196
_DNCHUNK = 98
_EPT = _CHUNK * _NCHUNK
_EPAD = _EPT * _NW
_BLKS = _EPAD // _LANE
_BPW = _BLKS // _NW
_NP = 102400
_PADR = _NP - _N
_NT = _NP
_RPT = _NT // _NTILE

_FOLD = 8
_NF = _NP // _FOLD
_NDEG = _NP // _LANE

_mesh = plsc.VectorSubcoreMesh(core_axis_name="c", subcore_axis_name="s")


def _deg_body(edges_hbm, out_hbm, didx, ones_v, zbuf, deg_acc, sem, sem_i):
    cid = lax.axis_index("c")
    sid = lax.axis_index("s")
    wid = sid * _NC + cid
    tid = sid

    def fill(i, c):
        ones_v[pl.ds(i * 16, 16)] = jnp.full((16,), 1.0, jnp.float32)
        return c

    lax.fori_loop(0, _DCHUNK // 16, fill, 0)

    def zfill(i, c):
        zbuf[pl.ds(i * 16, 16)] = jnp.zeros((16,), jnp.float32)
        return c

    lax.fori_loop(0, _RPT // 16, zfill, 0)
    pltpu.sync_copy(zbuf, deg_acc.at[pl.ds(tid * _RPT, _RPT)])
    plsc.subcore_barrier()

    base = wid * _BPW

    def load_idx(slot, c):
        pltpu.async_copy(
            edges_hbm.at[1, pl.ds(base + c * _DKB, _DKB)], didx.at[slot],
            sem_i)

    def wait_idx(slot):
        pltpu.make_async_copy(
            edges_hbm.at[1, pl.ds(base, _DKB)], didx.at[slot], sem_i).wait()

    load_idx(0, 0)

    def chunk(c, carry):
        b = lax.rem(c, 2)

        @pl.when(c + 1 < _DNCHUNK)
        def _prefetch():
            load_idx(1 - b, c + 1)

        wait_idx(b)
        handles = [
            pltpu.async_copy(
                ones_v.at[pl.ds(j * _LANE, _LANE)],
                deg_acc.at[didx.at[b, j]],
                sem,
                add=True,
            )
            for j in range(_DKB)
        ]
        for h in handles:
            h.wait()
        return carry

    lax.fori_loop(0, _DNCHUNK, chunk, 0)
    plsc.subcore_barrier()
    pltpu.sync_copy(
        deg_acc.at[pl.ds(tid * _RPT, _RPT)],
        out_hbm.at[cid, pl.ds(tid * _RPT, _RPT)],
    )


_SC_PARAMS = pltpu.CompilerParams(use_tc_tiling_on_sc=False)

_deg = pl.kernel(
    _deg_body,
    out_type=jax.ShapeDtypeStruct((_NC, _NT), jnp.float32),
    mesh=_mesh,
    compiler_params=_SC_PARAMS,
    scratch_types=[
        pltpu.VMEM((2, _DKB, _LANE), jnp.int32),
        pltpu.VMEM((_DCHUNK,), jnp.float32),
        pltpu.VMEM((_RPT,), jnp.float32),
        pltpu.VMEM_SHARED((_NT,), jnp.float32),
        pltpu.SemaphoreType.DMA,
        pltpu.SemaphoreType.DMA,
    ],
)


def _agg_body(g_hbm, edges_hbm, out_hbm, sidx, didx, rows, acc,
              sem_g, sem_s, sem_i):
    cid = lax.axis_index("c")
    sid = lax.axis_index("s")
    wid = sid * _NC + cid
    tid = sid

    def zrow(i, c):
        rows[0, i, :] = jnp.zeros((16,), jnp.float32)
        return c

    lax.fori_loop(0, _CHUNK, zrow, 0)
    for k in range(_RPT // _CHUNK):
        pltpu.sync_copy(rows.at[0],
                        acc.at[pl.ds(tid * _RPT + k * _CHUNK, _CHUNK)])
    _TAIL = _RPT % _CHUNK
    pltpu.sync_copy(
        rows.at[0, pl.ds(0, _TAIL)],
        acc.at[pl.ds(tid * _RPT + _RPT - _TAIL, _TAIL)],
    )
    plsc.subcore_barrier()

    base = wid * _BPW

    def load_idx(slot, c):
        row0 = base + c * _KB
        pltpu.async_copy(edges_hbm.at[0, pl.ds(row0, _KB)], sidx.at[slot],
                         sem_i)
        pltpu.async_copy(edges_hbm.at[1, pl.ds(row0, _KB)], didx.at[slot],
                         sem_i)

    def wait_idx(slot):
        pltpu.make_async_copy(
            edges_hbm.at[0, pl.ds(base, _KB)], sidx.at[slot], sem_i).wait()
        pltpu.make_async_copy(
            edges_hbm.at[1, pl.ds(base, _KB)], didx.at[slot], sem_i).wait()

    def fire_gather(buf, slot):
        for j in range(_KB):
            pltpu.async_copy(
                g_hbm.at[sidx.at[slot, j]],
                rows.at[buf, pl.ds(j * _LANE, _LANE)],
                sem_g,
            )

    def wait_gather(buf):
        for j in range(_KB):
            pltpu.make_async_copy(
                g_hbm.at[sidx.at[0, j]],
                rows.at[buf, pl.ds(j * _LANE, _LANE)],
                sem_g,
            ).wait()

    load_idx(0, 0)
    load_idx(1, 1)
    wait_idx(0)
    fire_gather(0, 0)
    wait_idx(1)
    fire_gather(1, 1)
    load_idx(2, 2)
    load_idx(3, 3)

    def chunk(c, carry):
        b = lax.rem(c, 2)
        q = lax.rem(c, 4)
        wait_gather(b)
        sh = [
            pltpu.async_copy(
                rows.at[b, pl.ds(j * _LANE, _LANE)],
                acc.at[didx.at[q, j]],
                sem_s,
                add=True,
            )
            for j in range(_KB)
        ]
        for h in sh:
            h.wait()

        @pl.when(c + 4 < _NCHUNK)
        def _prefetch():
            load_idx(q, c + 4)

        @pl.when(c + 2 < _NCHUNK)
        def _next_gather():
            qn = lax.rem(c + 2, 4)
            wait_idx(qn)
            fire_gather(b, qn)

        return carry

    lax.fori_loop(0, _NCHUNK, chunk, 0)
    plsc.subcore_barrier()
    pltpu.sync_copy(
        acc.at[pl.ds(tid * _RPT, _RPT)],
        out_hbm.at[cid, pl.ds(tid * _RPT, _RPT)],
    )


_agg = pl.kernel(
    _agg_body,
    out_type=jax.ShapeDtypeStruct((_NC, _NT, _D), jnp.float32),
    mesh=_mesh,
    compiler_params=_SC_PARAMS,
    scratch_types=[
        pltpu.VMEM((4, _KB, _LANE), jnp.int32),
        pltpu.VMEM((4, _KB, _LANE), jnp.int32),
        pltpu.VMEM((2, _CHUNK, _D), jnp.float32),
        pltpu.VMEM_SHARED((_NT, _D), jnp.float32),
        pltpu.SemaphoreType.DMA,
        pltpu.SemaphoreType.DMA,
        pltpu.SemaphoreType.DMA,
    ],
)


_RB = 256
_DB = _RB // _NTILE
_GRID = (_NF // _RB,)


def _expand_dinv(p0, p1):
    dinvp = lax.rsqrt(p0 + p1 + 1.0)
    rep = jnp.broadcast_to(dinvp[:, None, :], (_DB, 16, _LANE))
    rep = rep.reshape(_RB, _LANE)
    rr = lax.broadcasted_iota(jnp.int32, (_RB, _LANE), 0)
    cc = lax.broadcasted_iota(jnp.int32, (_RB, _LANE), 1)
    lmask = (cc // _FOLD == rr % 16).astype(jnp.float32)
    x1 = rep * lmask
    rc = lax.broadcasted_iota(jnp.int32, (_LANE, _LANE), 0)
    rl = lax.broadcasted_iota(jnp.int32, (_LANE, _LANE), 1)
    rmat = (rl // _D == rc % _FOLD).astype(jnp.float32)
    return jnp.dot(x1, rmat, preferred_element_type=jnp.float32,
                   precision=lax.Precision.HIGHEST)


def _tc1_body(x_ref, w1_ref, p0_ref, p1_ref, g_ref, dinv_ref):
    wide = _expand_dinv(p0_ref[0], p1_ref[0])
    h = jnp.dot(x_ref[...], w1_ref[...], preferred_element_type=jnp.float32)
    g_ref[...] = h * wide
    dinv_ref[...] = wide


def _tc2_body(a0_ref, a1_ref, g1_ref, dinv_ref, b1_ref, w2_ref, g2_ref):
    dinv = dinv_ref[...]
    z = (a0_ref[0] + a1_ref[0] + g1_ref[...]) * dinv + b1_ref[...]
    z = jnp.maximum(z, 0.0)
    g2_ref[...] = jnp.dot(z, w2_ref[...],
                          preferred_element_type=jnp.float32) * dinv


_RB3 = 256
_GRID3 = (49,)


def _tc3_body(a0_ref, a1_ref, g2_ref, dinv_ref, b2_ref, k1_ref, cb1_ref,
              k2_ref, cb2_ref, o1_ref, o2_ref):
    z = (a0_ref[0] + a1_ref[0] + g2_ref[...]) * dinv_ref[...] + b2_ref[...]
    z = jnp.maximum(z, 0.0)
    y = jnp.broadcast_to(z[:, None, :], (_RB3, _FOLD, _LANE))
    y = y.reshape(_RB3 * _FOLD, _LANE)
    n_i = lax.broadcasted_iota(jnp.int32, (_RB3 * _FOLD, _LANE), 0)
    l_i = lax.broadcasted_iota(jnp.int32, (_RB3 * _FOLD, _LANE), 1)
    ycom = y * (l_i // _D == n_i % _FOLD).astype(jnp.float32)
    o1_ref[...] = jnp.dot(ycom, k1_ref[...],
                          preferred_element_type=jnp.float32) + cb1_ref[...]
    o2_ref[...] = jnp.dot(ycom, k2_ref[...],
                          preferred_element_type=jnp.float32) + cb2_ref[...]


def _row_spec(w):
    return pl.BlockSpec((_RB, w), lambda i: (i, 0))


def _full_spec(h, w):
    return pl.BlockSpec((h, w), lambda i: (0, 0))


def _part_spec(p, h, w):
    return pl.BlockSpec((1, h, w), lambda i, _p=p: (_p, i, 0))


_tc1 = pl.pallas_call(
    _tc1_body,
    grid=_GRID,
    in_specs=[_row_spec(48), _full_spec(48, _LANE),
              _part_spec(0, _DB, _LANE), _part_spec(1, _DB, _LANE)],
    out_specs=[_row_spec(_LANE), _row_spec(_LANE)],
    out_shape=[
        jax.ShapeDtypeStruct((_NF, _LANE), jnp.float32),
        jax.ShapeDtypeStruct((_NF, _LANE), jnp.float32),
    ],
)

_tc2 = pl.pallas_call(
    _tc2_body,
    grid=_GRID,
    in_specs=[_part_spec(0, _RB, _LANE), _part_spec(1, _RB, _LANE),
              _row_spec(_LANE), _row_spec(_LANE),
              _full_spec(1, _LANE), _full_spec(_LANE, _LANE)],
    out_specs=[_row_spec(_LANE)],
    out_shape=[jax.ShapeDtypeStruct((_NF, _LANE), jnp.float32)],
)

_tc3 = pl.pallas_call(
    _tc3_body,
    grid=_GRID3,
    in_specs=[_part_spec(0, _RB3, _LANE), _part_spec(1, _RB3, _LANE),
              pl.BlockSpec((_RB3, _LANE), lambda i: (i, 0)),
              pl.BlockSpec((_RB3, _LANE), lambda i: (i, 0)),
              _full_spec(1, _LANE), _full_spec(_LANE, 13),
              _full_spec(1, 13), _full_spec(_LANE, 8),
              _full_spec(1, 8)],
    out_specs=[pl.BlockSpec((_RB3 * _FOLD, 13), lambda i: (i, 0)),
               pl.BlockSpec((_RB3 * _FOLD, 8), lambda i: (i, 0))],
    out_shape=[
        jax.ShapeDtypeStruct((_N, 13), jnp.float32),
        jax.ShapeDtypeStruct((_N, 8), jnp.float32),
    ],
)

_EYE8 = np.eye(_FOLD, dtype=np.float32)


def kernel(x, edge_index, W1, b1, W2, b2, CW1, Cb1, CW2, Cb2):
    pad_i = jnp.arange(_EPAD - _E, dtype=jnp.int32)
    pads = jnp.stack([pad_i % _N, _N + pad_i % _PADR])
    edges = jnp.concatenate([edge_index, pads],
                            axis=1).reshape(2, _BLKS, _LANE)

    xr = x.reshape(_N // _FOLD, _FOLD * 6)
    xf = jnp.concatenate(
        [xr, jnp.zeros((_NF - _N // _FOLD, _FOLD * 6), jnp.float32)])

    w1big = jnp.kron(_EYE8, W1)
    w2big = jnp.kron(_EYE8, W2)
    k1 = jnp.tile(CW1, (_FOLD, 1))
    k2 = jnp.tile(CW2, (_FOLD, 1))
    b1t = jnp.tile(b1, _FOLD).reshape(1, _LANE)
    b2t = jnp.tile(b2, _FOLD).reshape(1, _LANE)
    cb1s = Cb1.reshape(1, 13)
    cb2s = Cb2.reshape(1, 8)

    degp = _deg(edges)
    dg = degp.reshape(_NC, _NDEG, _LANE)

    g1f, dinvw = _tc1(xf, w1big, dg, dg)
    acc1 = _agg(g1f.reshape(_NT, _D), edges)
    (g2f,) = _tc2(acc1.reshape(_NC, _NF, _LANE),
                  acc1.reshape(_NC, _NF, _LANE),
                  g1f, dinvw, b1t, w2big)
    acc2 = _agg(g2f.reshape(_NT, _D), edges)
    out1, out2 = _tc3(acc2.reshape(_NC, _NF, _LANE),
                      acc2.reshape(_NC, _NF, _LANE),
                      g2f, dinvw, b2t, k1, cb1s, k2, cb2s)
    return (out1, out2)


# speedup vs baseline: 126.7663x; 1.1624x over previous
import jax
import jax.numpy as jnp
import numpy as np
from jax import lax
from jax.experimental import pallas as pl
from jax.experimental.pallas import tpu as pltpu
from jax.experimental.pallas import tpu_sc as plsc

_N = 100000
_E = 3200000
_D = 16
_LANE = 128
_KB = 4
_CHUNK = _LANE * _KB
_DKB = 8
_DCHUNK = _LANE * _DKB
_NC = 2
_NTILE = 16
_NW = _NC * _NTILE
_NCHUNK = 196
_DNCHUNK = 98
_EPT = _CHUNK * _NCHUNK
_EPAD = _EPT * _NW
_BLKS = _EPAD // _LANE
_BPW = _BLKS // _NW
_NP = 100352
_PADR = _NP - _N
_NT = _NP
_RPT = _NT // _NTILE

_FOLD = 8
_NF = _NP // _FOLD
_NDEG = _NP // _LANE

_mesh = plsc.VectorSubcoreMesh(core_axis_name="c", subcore_axis_name="s")


def _deg_body(edges_hbm, out_hbm, didx, ones_v, zbuf, deg_acc, sem, sem_i):
    cid = lax.axis_index("c")
    sid = lax.axis_index("s")
    wid = sid * _NC + cid
    tid = sid

    def fill(i, c):
        ones_v[pl.ds(i * 16, 16)] = jnp.full((16,), 1.0, jnp.float32)
        return c

    lax.fori_loop(0, _DCHUNK // 16, fill, 0)

    def zfill(i, c):
        zbuf[pl.ds(i * 16, 16)] = jnp.zeros((16,), jnp.float32)
        return c

    lax.fori_loop(0, _RPT // 16, zfill, 0)
    pltpu.sync_copy(zbuf, deg_acc.at[pl.ds(tid * _RPT, _RPT)])
    plsc.subcore_barrier()

    base = wid * _BPW

    def load_idx(slot, c):
        pltpu.async_copy(
            edges_hbm.at[1, pl.ds(base + c * _DKB, _DKB)], didx.at[slot],
            sem_i)

    def wait_idx(slot):
        pltpu.make_async_copy(
            edges_hbm.at[1, pl.ds(base, _DKB)], didx.at[slot], sem_i).wait()

    load_idx(0, 0)

    def chunk(c, carry):
        b = lax.rem(c, 2)

        @pl.when(c + 1 < _DNCHUNK)
        def _prefetch():
            load_idx(1 - b, c + 1)

        wait_idx(b)
        handles = [
            pltpu.async_copy(
                ones_v.at[pl.ds(j * _LANE, _LANE)],
                deg_acc.at[didx.at[b, j]],
                sem,
                add=True,
            )
            for j in range(_DKB)
        ]
        for h in handles:
            h.wait()
        return carry

    lax.fori_loop(0, _DNCHUNK, chunk, 0)
    plsc.subcore_barrier()
    pltpu.sync_copy(
        deg_acc.at[pl.ds(tid * _RPT, _RPT)],
        out_hbm.at[cid, pl.ds(tid * _RPT, _RPT)],
    )


_SC_PARAMS = pltpu.CompilerParams(use_tc_tiling_on_sc=False)

_deg = pl.kernel(
    _deg_body,
    out_type=jax.ShapeDtypeStruct((_NC, _NT), jnp.float32),
    mesh=_mesh,
    compiler_params=_SC_PARAMS,
    scratch_types=[
        pltpu.VMEM((2, _DKB, _LANE), jnp.int32),
        pltpu.VMEM((_DCHUNK,), jnp.float32),
        pltpu.VMEM((_RPT,), jnp.float32),
        pltpu.VMEM_SHARED((_NT,), jnp.float32),
        pltpu.SemaphoreType.DMA,
        pltpu.SemaphoreType.DMA,
    ],
)


def _agg_body(g_hbm, edges_hbm, out_hbm, sidx, didx, rows, acc,
              sem_g0, sem_g1, sem_s, sem_i0, sem_i1):
    cid = lax.axis_index("c")
    sid = lax.axis_index("s")
    wid = sid * _NC + cid
    tid = sid
    sem_g = (sem_g0, sem_g1)
    sem_i = (sem_i0, sem_i1)

    def zrow(i, c):
        rows[0, i, :] = jnp.zeros((16,), jnp.float32)
        return c

    lax.fori_loop(0, _CHUNK, zrow, 0)
    for k in range(_RPT // _CHUNK):
        pltpu.sync_copy(rows.at[0],
                        acc.at[pl.ds(tid * _RPT + k * _CHUNK, _CHUNK)])
    _TAIL = _RPT % _CHUNK
    pltpu.sync_copy(
        rows.at[0, pl.ds(0, _TAIL)],
        acc.at[pl.ds(tid * _RPT + _RPT - _TAIL, _TAIL)],
    )
    plsc.subcore_barrier()

    base = wid * _BPW

    def load_idx(slot, c, sem):
        row0 = base + c * _KB
        pltpu.async_copy(edges_hbm.at[0, pl.ds(row0, _KB)], sidx.at[slot],
                         sem)
        pltpu.async_copy(edges_hbm.at[1, pl.ds(row0, _KB)], didx.at[slot],
                         sem)

    def wait_idx(slot, sem):
        pltpu.make_async_copy(
            edges_hbm.at[0, pl.ds(base, _KB)], sidx.at[slot], sem).wait()
        pltpu.make_async_copy(
            edges_hbm.at[1, pl.ds(base, _KB)], didx.at[slot], sem).wait()

    def fire_gather(buf, slot, sem):
        for j in range(_KB):
            pltpu.async_copy(
                g_hbm.at[sidx.at[slot, j]],
                rows.at[buf, pl.ds(j * _LANE, _LANE)],
                sem,
            )

    def wait_gather(buf, sem):
        for j in range(_KB):
            pltpu.make_async_copy(
                g_hbm.at[sidx.at[0, j]],
                rows.at[buf, pl.ds(j * _LANE, _LANE)],
                sem,
            ).wait()

    def fire_scatter(buf, slot):
        for j in range(_KB):
            pltpu.async_copy(
                rows.at[buf, pl.ds(j * _LANE, _LANE)],
                acc.at[didx.at[slot, j]],
                sem_s,
                add=True,
            )

    def wait_scatter(buf, slot):
        for j in range(_KB):
            pltpu.make_async_copy(
                rows.at[buf, pl.ds(j * _LANE, _LANE)],
                acc.at[didx.at[slot, j]],
                sem_s,
            ).wait()

    pltpu.sync_copy(edges_hbm.at[0, pl.ds(base, _KB)], sidx.at[0])
    pltpu.sync_copy(edges_hbm.at[1, pl.ds(base, _KB)], didx.at[0])
    pltpu.sync_copy(edges_hbm.at[0, pl.ds(base + _KB, _KB)], sidx.at[1])
    pltpu.sync_copy(edges_hbm.at[1, pl.ds(base + _KB, _KB)], didx.at[1])
    fire_gather(0, 0, sem_g[0])
    fire_gather(1, 1, sem_g[1])
    load_idx(2, 2, sem_i[0])
    load_idx(3, 3, sem_i[1])

    def quad(i, carry):
        for k in range(4):
            c = 4 * i + k
            bcur = lax.rem(c, 3)

            @pl.when(c > 0)
            def _drain_prev():
                wait_scatter(lax.rem(c + 2, 3), (k + 3) % 4)

            @pl.when((c > 0) & (c + 3 < _NCHUNK))
            def _prefetch_idx():
                load_idx((k + 3) % 4, c + 3, sem_i[(k + 1) % 2])

            wait_gather(bcur, sem_g[k % 2])
            fire_scatter(bcur, k)

            @pl.when(c + 2 < _NCHUNK)
            def _next_gather():
                wait_idx((k + 2) % 4, sem_i[k % 2])
                fire_gather(lax.rem(c + 2, 3), (k + 2) % 4, sem_g[k % 2])

        return carry

    lax.fori_loop(0, _NCHUNK // 4, quad, 0)
    wait_scatter(lax.rem(_NCHUNK - 1, 3), (_NCHUNK - 1) % 4)
    plsc.subcore_barrier()
    pltpu.sync_copy(
        acc.at[pl.ds(tid * _RPT, _RPT)],
        out_hbm.at[cid, pl.ds(tid * _RPT, _RPT)],
    )


_agg = pl.kernel(
    _agg_body,
    out_type=jax.ShapeDtypeStruct((_NC, _NT, _D), jnp.float32),
    mesh=_mesh,
    compiler_params=_SC_PARAMS,
    scratch_types=[
        pltpu.VMEM((4, _KB, _LANE), jnp.int32),
        pltpu.VMEM((4, _KB, _LANE), jnp.int32),
        pltpu.VMEM((3, _CHUNK, _D), jnp.float32),
        pltpu.VMEM_SHARED((_NT, _D), jnp.float32),
        pltpu.SemaphoreType.DMA,
        pltpu.SemaphoreType.DMA,
        pltpu.SemaphoreType.DMA,
        pltpu.SemaphoreType.DMA,
        pltpu.SemaphoreType.DMA,
    ],
)


_RB = 256
_DB = _RB // _NTILE
_GRID = (_NF // _RB,)


def _expand_dinv(p0, p1):
    dinvp = lax.rsqrt(p0 + p1 + 1.0)
    rep = jnp.broadcast_to(dinvp[:, None, :], (_DB, 16, _LANE))
    rep = rep.reshape(_RB, _LANE)
    rr = lax.broadcasted_iota(jnp.int32, (_RB, _LANE), 0)
    cc = lax.broadcasted_iota(jnp.int32, (_RB, _LANE), 1)
    lmask = (cc // _FOLD == rr % 16).astype(jnp.float32)
    x1 = rep * lmask
    rc = lax.broadcasted_iota(jnp.int32, (_LANE, _LANE), 0)
    rl = lax.broadcasted_iota(jnp.int32, (_LANE, _LANE), 1)
    rmat = (rl // _D == rc % _FOLD).astype(jnp.float32)
    return jnp.dot(x1, rmat, preferred_element_type=jnp.float32,
                   precision=lax.Precision.HIGHEST)


def _tc1_body(x_ref, w1_ref, p0_ref, p1_ref, g_ref, dinv_ref):
    wide = _expand_dinv(p0_ref[0], p1_ref[0])
    h = jnp.dot(x_ref[...], w1_ref[...], preferred_element_type=jnp.float32)
    g_ref[...] = h * wide
    dinv_ref[...] = wide


def _tc2_body(a0_ref, a1_ref, g1_ref, dinv_ref, b1_ref, w2_ref, g2_ref):
    dinv = dinv_ref[...]
    z = (a0_ref[0] + a1_ref[0] + g1_ref[...]) * dinv + b1_ref[...]
    z = jnp.maximum(z, 0.0)
    g2_ref[...] = jnp.dot(z, w2_ref[...],
                          preferred_element_type=jnp.float32) * dinv


_RB3 = 256
_GRID3 = (49,)


def _tc3_body(a0_ref, a1_ref, g2_ref, dinv_ref, b2_ref, k1_ref, cb1_ref,
              k2_ref, cb2_ref, o1_ref, o2_ref):
    z = (a0_ref[0] + a1_ref[0] + g2_ref[...]) * dinv_ref[...] + b2_ref[...]
    z = jnp.maximum(z, 0.0)
    y = jnp.broadcast_to(z[:, None, :], (_RB3, _FOLD, _LANE))
    y = y.reshape(_RB3 * _FOLD, _LANE)
    n_i = lax.broadcasted_iota(jnp.int32, (_RB3 * _FOLD, _LANE), 0)
    l_i = lax.broadcasted_iota(jnp.int32, (_RB3 * _FOLD, _LANE), 1)
    ycom = y * (l_i // _D == n_i % _FOLD).astype(jnp.float32)
    o1_ref[...] = jnp.dot(ycom, k1_ref[...],
                          preferred_element_type=jnp.float32) + cb1_ref[...]
    o2_ref[...] = jnp.dot(ycom, k2_ref[...],
                          preferred_element_type=jnp.float32) + cb2_ref[...]


def _row_spec(w):
    return pl.BlockSpec((_RB, w), lambda i: (i, 0))


def _full_spec(h, w):
    return pl.BlockSpec((h, w), lambda i: (0, 0))


def _part_spec(p, h, w):
    return pl.BlockSpec((1, h, w), lambda i, _p=p: (_p, i, 0))


_tc1 = pl.pallas_call(
    _tc1_body,
    grid=_GRID,
    in_specs=[_row_spec(48), _full_spec(48, _LANE),
              _part_spec(0, _DB, _LANE), _part_spec(1, _DB, _LANE)],
    out_specs=[_row_spec(_LANE), _row_spec(_LANE)],
    out_shape=[
        jax.ShapeDtypeStruct((_NF, _LANE), jnp.float32),
        jax.ShapeDtypeStruct((_NF, _LANE), jnp.float32),
    ],
)

_tc2 = pl.pallas_call(
    _tc2_body,
    grid=_GRID,
    in_specs=[_part_spec(0, _RB, _LANE), _part_spec(1, _RB, _LANE),
              _row_spec(_LANE), _row_spec(_LANE),
              _full_spec(1, _LANE), _full_spec(_LANE, _LANE)],
    out_specs=[_row_spec(_LANE)],
    out_shape=[jax.ShapeDtypeStruct((_NF, _LANE), jnp.float32)],
)

_tc3 = pl.pallas_call(
    _tc3_body,
    grid=_GRID3,
    in_specs=[_part_spec(0, _RB3, _LANE), _part_spec(1, _RB3, _LANE),
              pl.BlockSpec((_RB3, _LANE), lambda i: (i, 0)),
              pl.BlockSpec((_RB3, _LANE), lambda i: (i, 0)),
              _full_spec(1, _LANE), _full_spec(_LANE, 13),
              _full_spec(1, 13), _full_spec(_LANE, 8),
              _full_spec(1, 8)],
    out_specs=[pl.BlockSpec((_RB3 * _FOLD, 13), lambda i: (i, 0)),
               pl.BlockSpec((_RB3 * _FOLD, 8), lambda i: (i, 0))],
    out_shape=[
        jax.ShapeDtypeStruct((_N, 13), jnp.float32),
        jax.ShapeDtypeStruct((_N, 8), jnp.float32),
    ],
)

_EYE8 = np.eye(_FOLD, dtype=np.float32)


def kernel(x, edge_index, W1, b1, W2, b2, CW1, Cb1, CW2, Cb2):
    pad_i = jnp.arange(_EPAD - _E, dtype=jnp.int32)
    pads = jnp.stack([pad_i % _N, _N + pad_i % _PADR])
    edges = jnp.concatenate([edge_index, pads],
                            axis=1).reshape(2, _BLKS, _LANE)

    xr = x.reshape(_N // _FOLD, _FOLD * 6)
    xf = jnp.concatenate(
        [xr, jnp.zeros((_NF - _N // _FOLD, _FOLD * 6), jnp.float32)])

    w1big = jnp.kron(_EYE8, W1)
    w2big = jnp.kron(_EYE8, W2)
    k1 = jnp.tile(CW1, (_FOLD, 1))
    k2 = jnp.tile(CW2, (_FOLD, 1))
    b1t = jnp.tile(b1, _FOLD).reshape(1, _LANE)
    b2t = jnp.tile(b2, _FOLD).reshape(1, _LANE)
    cb1s = Cb1.reshape(1, 13)
    cb2s = Cb2.reshape(1, 8)

    degp = _deg(edges)
    dg = degp.reshape(_NC, _NDEG, _LANE)

    g1f, dinvw = _tc1(xf, w1big, dg, dg)
    acc1 = _agg(g1f.reshape(_NT, _D), edges)
    (g2f,) = _tc2(acc1.reshape(_NC, _NF, _LANE),
                  acc1.reshape(_NC, _NF, _LANE),
                  g1f, dinvw, b1t, w2big)
    acc2 = _agg(g2f.reshape(_NT, _D), edges)
    out1, out2 = _tc3(acc2.reshape(_NC, _NF, _LANE),
                      acc2.reshape(_NC, _NF, _LANE),
                      g2f, dinvw, b2t, k1, cb1s, k2, cb2s)
    return (out1, out2)


# speedup vs baseline: 135.6116x; 1.0698x over previous
import jax
import jax.numpy as jnp
import numpy as np
from jax import lax
from jax.experimental import pallas as pl
from jax.experimental.pallas import tpu as pltpu
from jax.experimental.pallas import tpu_sc as plsc

_N = 100000
_E = 3200000
_D = 16
_LANE = 128
_KB = 4
_CHUNK = _LANE * _KB
_DKB = 8
_DCHUNK = _LANE * _DKB
_NC = 2
_NTILE = 16
_NW = _NC * _NTILE
_NCHUNK = 196
_DNCHUNK = 98
_EPT = _CHUNK * _NCHUNK
_EPAD = _EPT * _NW
_BLKS = _EPAD // _LANE
_BPW = _BLKS // _NW
_NP = 100352
_PADR = _NP - _N
_NT = _NP
_RPT = _NT // _NTILE

_FOLD = 8
_NF = _NP // _FOLD
_NDEG = _NP // _LANE

_mesh = plsc.VectorSubcoreMesh(core_axis_name="c", subcore_axis_name="s")


def _deg_body(dst_hbm, out_hbm, didx, ones_v, zbuf, deg_acc, sem, sem_i):
    cid = lax.axis_index("c")
    sid = lax.axis_index("s")
    wid = sid * _NC + cid
    tid = sid

    def fill(i, c):
        ones_v[pl.ds(i * 16, 16)] = jnp.full((16,), 1.0, jnp.float32)
        return c

    lax.fori_loop(0, _DCHUNK // 16, fill, 0)

    def zfill(i, c):
        zbuf[pl.ds(i * 16, 16)] = jnp.zeros((16,), jnp.float32)
        return c

    lax.fori_loop(0, _RPT // 16, zfill, 0)
    pltpu.sync_copy(zbuf, deg_acc.at[pl.ds(tid * _RPT, _RPT)])
    plsc.subcore_barrier()

    base = wid * _BPW

    def load_idx(slot, c):
        pltpu.async_copy(
            dst_hbm.at[pl.ds(base + c * _DKB, _DKB)], didx.at[slot], sem_i)

    def wait_idx(slot):
        pltpu.make_async_copy(
            dst_hbm.at[pl.ds(base, _DKB)], didx.at[slot], sem_i).wait()

    load_idx(0, 0)

    def chunk(c, carry):
        b = lax.rem(c, 2)

        @pl.when(c + 1 < _DNCHUNK)
        def _prefetch():
            load_idx(1 - b, c + 1)

        wait_idx(b)
        handles = [
            pltpu.async_copy(
                ones_v.at[pl.ds(j * _LANE, _LANE)],
                deg_acc.at[didx.at[b, j]],
                sem,
                add=True,
            )
            for j in range(_DKB)
        ]
        for h in handles:
            h.wait()
        return carry

    lax.fori_loop(0, _DNCHUNK, chunk, 0)
    plsc.subcore_barrier()
    pltpu.sync_copy(
        deg_acc.at[pl.ds(tid * _RPT, _RPT)],
        out_hbm.at[cid, pl.ds(tid * _RPT, _RPT)],
    )


_SC_PARAMS = pltpu.CompilerParams(use_tc_tiling_on_sc=False)

_deg = pl.kernel(
    _deg_body,
    out_type=jax.ShapeDtypeStruct((_NC, _NT), jnp.float32),
    mesh=_mesh,
    compiler_params=_SC_PARAMS,
    scratch_types=[
        pltpu.VMEM((2, _DKB, _LANE), jnp.int32),
        pltpu.VMEM((_DCHUNK,), jnp.float32),
        pltpu.VMEM((_RPT,), jnp.float32),
        pltpu.VMEM_SHARED((_NT,), jnp.float32),
        pltpu.SemaphoreType.DMA,
        pltpu.SemaphoreType.DMA,
    ],
)


def _agg_body(g_hbm, src_hbm, dst_hbm, out_hbm, sidx, didx, rows, acc,
              sem_g0, sem_g1, sem_s, sem_i0, sem_i1):
    cid = lax.axis_index("c")
    sid = lax.axis_index("s")
    wid = sid * _NC + cid
    tid = sid
    sem_g = (sem_g0, sem_g1)
    sem_i = (sem_i0, sem_i1)

    def zrow(i, c):
        rows[0, i, :] = jnp.zeros((16,), jnp.float32)
        return c

    lax.fori_loop(0, _CHUNK, zrow, 0)
    for k in range(_RPT // _CHUNK):
        pltpu.sync_copy(rows.at[0],
                        acc.at[pl.ds(tid * _RPT + k * _CHUNK, _CHUNK)])
    _TAIL = _RPT % _CHUNK
    pltpu.sync_copy(
        rows.at[0, pl.ds(0, _TAIL)],
        acc.at[pl.ds(tid * _RPT + _RPT - _TAIL, _TAIL)],
    )
    plsc.subcore_barrier()

    base = wid * _BPW

    def load_idx(slot, c, sem):
        row0 = base + c * _KB
        pltpu.async_copy(src_hbm.at[pl.ds(row0, _KB)], sidx.at[slot], sem)
        pltpu.async_copy(dst_hbm.at[pl.ds(row0, _KB)], didx.at[slot], sem)

    def wait_idx(slot, sem):
        pltpu.make_async_copy(
            src_hbm.at[pl.ds(base, _KB)], sidx.at[slot], sem).wait()
        pltpu.make_async_copy(
            dst_hbm.at[pl.ds(base, _KB)], didx.at[slot], sem).wait()

    def fire_gather(buf, slot, sem):
        for j in range(_KB):
            pltpu.async_copy(
                g_hbm.at[sidx.at[slot, j]],
                rows.at[buf, pl.ds(j * _LANE, _LANE)],
                sem,
            )

    def wait_gather(buf, sem):
        for j in range(_KB):
            pltpu.make_async_copy(
                g_hbm.at[sidx.at[0, j]],
                rows.at[buf, pl.ds(j * _LANE, _LANE)],
                sem,
            ).wait()

    def fire_scatter(buf, slot):
        for j in range(_KB):
            pltpu.async_copy(
                rows.at[buf, pl.ds(j * _LANE, _LANE)],
                acc.at[didx.at[slot, j]],
                sem_s,
                add=True,
            )

    def wait_scatter(buf, slot):
        for j in range(_KB):
            pltpu.make_async_copy(
                rows.at[buf, pl.ds(j * _LANE, _LANE)],
                acc.at[didx.at[slot, j]],
                sem_s,
            ).wait()

    pltpu.sync_copy(src_hbm.at[pl.ds(base, _KB)], sidx.at[0])
    pltpu.sync_copy(dst_hbm.at[pl.ds(base, _KB)], didx.at[0])
    pltpu.sync_copy(src_hbm.at[pl.ds(base + _KB, _KB)], sidx.at[1])
    pltpu.sync_copy(dst_hbm.at[pl.ds(base + _KB, _KB)], didx.at[1])
    fire_gather(0, 0, sem_g[0])
    fire_gather(1, 1, sem_g[1])
    load_idx(2, 2, sem_i[0])
    load_idx(3, 3, sem_i[1])

    def quad(i, carry):
        for k in range(4):
            c = 4 * i + k
            bcur = lax.rem(c, 3)

            @pl.when(c > 0)
            def _drain_prev():
                wait_scatter(lax.rem(c + 2, 3), (k + 3) % 4)

            @pl.when((c > 0) & (c + 3 < _NCHUNK))
            def _prefetch_idx():
                load_idx((k + 3) % 4, c + 3, sem_i[(k + 1) % 2])

            wait_gather(bcur, sem_g[k % 2])
            fire_scatter(bcur, k)

            @pl.when(c + 2 < _NCHUNK)
            def _next_gather():
                wait_idx((k + 2) % 4, sem_i[k % 2])
                fire_gather(lax.rem(c + 2, 3), (k + 2) % 4, sem_g[k % 2])

        return carry

    lax.fori_loop(0, _NCHUNK // 4, quad, 0)
    wait_scatter(lax.rem(_NCHUNK - 1, 3), (_NCHUNK - 1) % 4)
    plsc.subcore_barrier()
    pltpu.sync_copy(
        acc.at[pl.ds(tid * _RPT, _RPT)],
        out_hbm.at[cid, pl.ds(tid * _RPT, _RPT)],
    )


_agg = pl.kernel(
    _agg_body,
    out_type=jax.ShapeDtypeStruct((_NC, _NT, _D), jnp.float32),
    mesh=_mesh,
    compiler_params=_SC_PARAMS,
    scratch_types=[
        pltpu.VMEM((4, _KB, _LANE), jnp.int32),
        pltpu.VMEM((4, _KB, _LANE), jnp.int32),
        pltpu.VMEM((3, _CHUNK, _D), jnp.float32),
        pltpu.VMEM_SHARED((_NT, _D), jnp.float32),
        pltpu.SemaphoreType.DMA,
        pltpu.SemaphoreType.DMA,
        pltpu.SemaphoreType.DMA,
        pltpu.SemaphoreType.DMA,
        pltpu.SemaphoreType.DMA,
    ],
)


_RB = 256
_DB = _RB // _NTILE
_GRID = (_NF // _RB,)


def _expand_dinv(p0, p1):
    dinvp = lax.rsqrt(p0 + p1 + 1.0)
    rep = jnp.broadcast_to(dinvp[:, None, :], (_DB, 16, _LANE))
    rep = rep.reshape(_RB, _LANE)
    rr = lax.broadcasted_iota(jnp.int32, (_RB, _LANE), 0)
    cc = lax.broadcasted_iota(jnp.int32, (_RB, _LANE), 1)
    lmask = (cc // _FOLD == rr % 16).astype(jnp.float32)
    x1 = rep * lmask
    rc = lax.broadcasted_iota(jnp.int32, (_LANE, _LANE), 0)
    rl = lax.broadcasted_iota(jnp.int32, (_LANE, _LANE), 1)
    rmat = (rl // _D == rc % _FOLD).astype(jnp.float32)
    return jnp.dot(x1, rmat, preferred_element_type=jnp.float32,
                   precision=lax.Precision.HIGHEST)


def _tc1_body(x_ref, w1_ref, p0_ref, p1_ref, g_ref, dinv_ref):
    wide = _expand_dinv(p0_ref[0], p1_ref[0])
    h = jnp.dot(x_ref[...], w1_ref[...], preferred_element_type=jnp.float32)
    g_ref[...] = h * wide
    dinv_ref[...] = wide


def _tc2_body(a0_ref, a1_ref, g1_ref, dinv_ref, b1_ref, w2_ref, g2_ref):
    dinv = dinv_ref[...]
    z = (a0_ref[0] + a1_ref[0] + g1_ref[...]) * dinv + b1_ref[...]
    z = jnp.maximum(z, 0.0)
    g2_ref[...] = jnp.dot(z, w2_ref[...],
                          preferred_element_type=jnp.float32) * dinv


_RB3 = 256
_GRID3 = (49,)


def _tc3_body(a0_ref, a1_ref, g2_ref, dinv_ref, b2_ref, k1_ref, cb1_ref,
              k2_ref, cb2_ref, o1_ref, o2_ref):
    z = (a0_ref[0] + a1_ref[0] + g2_ref[...]) * dinv_ref[...] + b2_ref[...]
    z = jnp.maximum(z, 0.0)
    y = jnp.broadcast_to(z[:, None, :], (_RB3, _FOLD, _LANE))
    y = y.reshape(_RB3 * _FOLD, _LANE)
    n_i = lax.broadcasted_iota(jnp.int32, (_RB3 * _FOLD, _LANE), 0)
    l_i = lax.broadcasted_iota(jnp.int32, (_RB3 * _FOLD, _LANE), 1)
    ycom = y * (l_i // _D == n_i % _FOLD).astype(jnp.float32)
    o1_ref[...] = jnp.dot(ycom, k1_ref[...],
                          preferred_element_type=jnp.float32) + cb1_ref[...]
    o2_ref[...] = jnp.dot(ycom, k2_ref[...],
                          preferred_element_type=jnp.float32) + cb2_ref[...]


def _row_spec(w):
    return pl.BlockSpec((_RB, w), lambda i: (i, 0))


def _full_spec(h, w):
    return pl.BlockSpec((h, w), lambda i: (0, 0))


def _part_spec(p, h, w):
    return pl.BlockSpec((1, h, w), lambda i, _p=p: (_p, i, 0))


_tc1 = pl.pallas_call(
    _tc1_body,
    grid=_GRID,
    in_specs=[_row_spec(48), _full_spec(48, _LANE),
              _part_spec(0, _DB, _LANE), _part_spec(1, _DB, _LANE)],
    out_specs=[_row_spec(_LANE), _row_spec(_LANE)],
    out_shape=[
        jax.ShapeDtypeStruct((_NF, _LANE), jnp.float32),
        jax.ShapeDtypeStruct((_NF, _LANE), jnp.float32),
    ],
)

_tc2 = pl.pallas_call(
    _tc2_body,
    grid=_GRID,
    in_specs=[_part_spec(0, _RB, _LANE), _part_spec(1, _RB, _LANE),
              _row_spec(_LANE), _row_spec(_LANE),
              _full_spec(1, _LANE), _full_spec(_LANE, _LANE)],
    out_specs=[_row_spec(_LANE)],
    out_shape=[jax.ShapeDtypeStruct((_NF, _LANE), jnp.float32)],
)

_tc3 = pl.pallas_call(
    _tc3_body,
    grid=_GRID3,
    in_specs=[_part_spec(0, _RB3, _LANE), _part_spec(1, _RB3, _LANE),
              pl.BlockSpec((_RB3, _LANE), lambda i: (i, 0)),
              pl.BlockSpec((_RB3, _LANE), lambda i: (i, 0)),
              _full_spec(1, _LANE), _full_spec(_LANE, 13),
              _full_spec(1, 13), _full_spec(_LANE, 8),
              _full_spec(1, 8)],
    out_specs=[pl.BlockSpec((_RB3 * _FOLD, 13), lambda i: (i, 0)),
               pl.BlockSpec((_RB3 * _FOLD, 8), lambda i: (i, 0))],
    out_shape=[
        jax.ShapeDtypeStruct((_N, 13), jnp.float32),
        jax.ShapeDtypeStruct((_N, 8), jnp.float32),
    ],
)

_EYE8 = np.eye(_FOLD, dtype=np.float32)


def kernel(x, edge_index, W1, b1, W2, b2, CW1, Cb1, CW2, Cb2):
    pad_i = jnp.arange(_EPAD - _E, dtype=jnp.int32)
    src2d = jnp.concatenate([edge_index[0],
                             pad_i % _N]).reshape(_BLKS, _LANE)
    dst2d = jnp.concatenate([edge_index[1],
                             _N + pad_i % _PADR]).reshape(_BLKS, _LANE)

    xr = x.reshape(_N // _FOLD, _FOLD * 6)
    xf = jnp.concatenate(
        [xr, jnp.zeros((_NF - _N // _FOLD, _FOLD * 6), jnp.float32)])

    w1big = jnp.kron(_EYE8, W1)
    w2big = jnp.kron(_EYE8, W2)
    k1 = jnp.tile(CW1, (_FOLD, 1))
    k2 = jnp.tile(CW2, (_FOLD, 1))
    b1t = jnp.tile(b1, _FOLD).reshape(1, _LANE)
    b2t = jnp.tile(b2, _FOLD).reshape(1, _LANE)
    cb1s = Cb1.reshape(1, 13)
    cb2s = Cb2.reshape(1, 8)

    degp = _deg(dst2d)
    dg = degp.reshape(_NC, _NDEG, _LANE)

    g1f, dinvw = _tc1(xf, w1big, dg, dg)
    acc1 = _agg(g1f.reshape(_NT, _D), src2d, dst2d)
    (g2f,) = _tc2(acc1.reshape(_NC, _NF, _LANE),
                  acc1.reshape(_NC, _NF, _LANE),
                  g1f, dinvw, b1t, w2big)
    acc2 = _agg(g2f.reshape(_NT, _D), src2d, dst2d)
    out1, out2 = _tc3(acc2.reshape(_NC, _NF, _LANE),
                      acc2.reshape(_NC, _NF, _LANE),
                      g2f, dinvw, b2t, k1, cb1s, k2, cb2s)
    return (out1, out2)


# speedup vs baseline: 135.6818x; 1.0005x over previous
import jax
import jax.numpy as jnp
import numpy as np
from jax import lax
from jax.experimental import pallas as pl
from jax.experimental.pallas import tpu as pltpu
from jax.experimental.pallas import tpu_sc as plsc

_N = 100000
_E = 3200000
_D = 16
_LANE = 128
_KB = 4
_CHUNK = _LANE * _KB
_DKB = 8
_DCHUNK = _LANE * _DKB
_NC = 2
_NTILE = 16
_NW = _NC * _NTILE
_NCHUNK = 196
_DNCHUNK = 98
_EPT = _CHUNK * _NCHUNK
_EPAD = _EPT * _NW
_BLKS = _EPAD // _LANE
_BPW = _BLKS // _NW
_NP = 100352
_PADR = _NP - _N
_NT = _NP
_RPT = _NT // _NTILE

_FOLD = 8
_NF = _NP // _FOLD
_NDEG = _NP // _LANE

_mesh = plsc.VectorSubcoreMesh(core_axis_name="c", subcore_axis_name="s")


def _deg_body(dst_hbm, out_hbm, didx, ones_v, zbuf, deg_acc, sem, sem_i):
    cid = lax.axis_index("c")
    sid = lax.axis_index("s")
    wid = sid * _NC + cid
    tid = sid

    def fill(i, c):
        ones_v[pl.ds(i * 16, 16)] = jnp.full((16,), 1.0, jnp.float32)
        return c

    lax.fori_loop(0, _DCHUNK // 16, fill, 0)

    def zfill(i, c):
        zbuf[pl.ds(i * 16, 16)] = jnp.zeros((16,), jnp.float32)
        return c

    lax.fori_loop(0, _RPT // 16, zfill, 0)
    pltpu.sync_copy(zbuf, deg_acc.at[pl.ds(tid * _RPT, _RPT)])
    plsc.subcore_barrier()

    base = wid * _BPW

    def load_idx(slot, c):
        pltpu.async_copy(
            dst_hbm.at[pl.ds(base + c * _DKB, _DKB)], didx.at[slot], sem_i)

    def wait_idx(slot):
        pltpu.make_async_copy(
            dst_hbm.at[pl.ds(base, _DKB)], didx.at[slot], sem_i).wait()

    load_idx(0, 0)

    def chunk(c, carry):
        b = lax.rem(c, 2)

        @pl.when(c + 1 < _DNCHUNK)
        def _prefetch():
            load_idx(1 - b, c + 1)

        wait_idx(b)
        handles = [
            pltpu.async_copy(
                ones_v.at[pl.ds(j * _LANE, _LANE)],
                deg_acc.at[didx.at[b, j]],
                sem,
                add=True,
            )
            for j in range(_DKB)
        ]
        for h in handles:
            h.wait()
        return carry

    lax.fori_loop(0, _DNCHUNK, chunk, 0)
    plsc.subcore_barrier()
    pltpu.sync_copy(
        deg_acc.at[pl.ds(tid * _RPT, _RPT)],
        out_hbm.at[cid, pl.ds(tid * _RPT, _RPT)],
    )


_SC_PARAMS = pltpu.CompilerParams(use_tc_tiling_on_sc=False)

_deg = pl.kernel(
    _deg_body,
    out_type=jax.ShapeDtypeStruct((_NC, _NT), jnp.float32),
    mesh=_mesh,
    compiler_params=_SC_PARAMS,
    scratch_types=[
        pltpu.VMEM((2, _DKB, _LANE), jnp.int32),
        pltpu.VMEM((_DCHUNK,), jnp.float32),
        pltpu.VMEM((_RPT,), jnp.float32),
        pltpu.VMEM_SHARED((_NT,), jnp.float32),
        pltpu.SemaphoreType.DMA,
        pltpu.SemaphoreType.DMA,
    ],
)


def _agg_body(g_hbm, src_hbm, dst_hbm, out_hbm, sidx, didx, rows, acc,
              sem_g0, sem_g1, sem_s, sem_i0, sem_i1):
    cid = lax.axis_index("c")
    sid = lax.axis_index("s")
    wid = sid * _NC + cid
    tid = sid
    sem_g = (sem_g0, sem_g1)
    sem_i = (sem_i0, sem_i1)

    def zrow(i, c):
        rows[0, i, :] = jnp.zeros((16,), jnp.float32)
        return c

    lax.fori_loop(0, _CHUNK, zrow, 0)
    for k in range(_RPT // _CHUNK):
        pltpu.sync_copy(rows.at[0],
                        acc.at[pl.ds(tid * _RPT + k * _CHUNK, _CHUNK)])
    _TAIL = _RPT % _CHUNK
    pltpu.sync_copy(
        rows.at[0, pl.ds(0, _TAIL)],
        acc.at[pl.ds(tid * _RPT + _RPT - _TAIL, _TAIL)],
    )
    plsc.subcore_barrier()

    base = wid * _BPW

    def load_idx(slot, c, sem):
        row0 = base + c * _KB
        pltpu.async_copy(src_hbm.at[pl.ds(row0, _KB)], sidx.at[slot], sem)
        pltpu.async_copy(dst_hbm.at[pl.ds(row0, _KB)], didx.at[slot], sem)

    def wait_idx(slot, sem):
        pltpu.make_async_copy(
            src_hbm.at[pl.ds(base, _KB)], sidx.at[slot], sem).wait()
        pltpu.make_async_copy(
            dst_hbm.at[pl.ds(base, _KB)], didx.at[slot], sem).wait()

    def fire_gather(buf, slot, sem):
        for j in range(_KB):
            pltpu.async_copy(
                g_hbm.at[sidx.at[slot, j]],
                rows.at[buf, pl.ds(j * _LANE, _LANE)],
                sem,
            )

    def wait_gather(buf, sem):
        for j in range(_KB):
            pltpu.make_async_copy(
                g_hbm.at[sidx.at[0, j]],
                rows.at[buf, pl.ds(j * _LANE, _LANE)],
                sem,
            ).wait()

    def fire_scatter(buf, slot):
        for j in range(_KB):
            pltpu.async_copy(
                rows.at[buf, pl.ds(j * _LANE, _LANE)],
                acc.at[didx.at[slot, j]],
                sem_s,
                add=True,
            )

    def wait_scatter(buf, slot):
        for j in range(_KB):
            pltpu.make_async_copy(
                rows.at[buf, pl.ds(j * _LANE, _LANE)],
                acc.at[didx.at[slot, j]],
                sem_s,
            ).wait()

    pltpu.sync_copy(src_hbm.at[pl.ds(base, _KB)], sidx.at[0])
    pltpu.sync_copy(dst_hbm.at[pl.ds(base, _KB)], didx.at[0])
    pltpu.sync_copy(src_hbm.at[pl.ds(base + _KB, _KB)], sidx.at[1])
    pltpu.sync_copy(dst_hbm.at[pl.ds(base + _KB, _KB)], didx.at[1])
    fire_gather(0, 0, sem_g[0])
    fire_gather(1, 1, sem_g[1])
    load_idx(2, 2, sem_i[0])
    load_idx(3, 3, sem_i[1])

    def quad(i, carry):
        for k in range(4):
            c = 4 * i + k
            bcur = lax.rem(c, 3)

            @pl.when(c > 0)
            def _drain_prev():
                wait_scatter(lax.rem(c + 2, 3), (k + 3) % 4)

            @pl.when((c > 0) & (c + 3 < _NCHUNK))
            def _prefetch_idx():
                load_idx((k + 3) % 4, c + 3, sem_i[(k + 1) % 2])

            wait_gather(bcur, sem_g[k % 2])
            fire_scatter(bcur, k)

            @pl.when(c + 2 < _NCHUNK)
            def _next_gather():
                wait_idx((k + 2) % 4, sem_i[k % 2])
                fire_gather(lax.rem(c + 2, 3), (k + 2) % 4, sem_g[k % 2])

        return carry

    lax.fori_loop(0, _NCHUNK // 4, quad, 0)
    wait_scatter(lax.rem(_NCHUNK - 1, 3), (_NCHUNK - 1) % 4)
    plsc.subcore_barrier()
    pltpu.sync_copy(
        acc.at[pl.ds(tid * _RPT, _RPT)],
        out_hbm.at[cid, pl.ds(tid * _RPT, _RPT)],
    )


_agg = pl.kernel(
    _agg_body,
    out_type=jax.ShapeDtypeStruct((_NC, _NT, _D), jnp.float32),
    mesh=_mesh,
    compiler_params=_SC_PARAMS,
    scratch_types=[
        pltpu.VMEM((4, _KB, _LANE), jnp.int32),
        pltpu.VMEM((4, _KB, _LANE), jnp.int32),
        pltpu.VMEM((3, _CHUNK, _D), jnp.float32),
        pltpu.VMEM_SHARED((_NT, _D), jnp.float32),
        pltpu.SemaphoreType.DMA,
        pltpu.SemaphoreType.DMA,
        pltpu.SemaphoreType.DMA,
        pltpu.SemaphoreType.DMA,
        pltpu.SemaphoreType.DMA,
    ],
)


_RB = 256
_DB = _RB // _NTILE
_GRID = (_NF // _RB,)


def _expand_dinv(p0, p1):
    dinvp = lax.rsqrt(p0 + p1 + 1.0)
    rep = jnp.broadcast_to(dinvp[:, None, :], (_DB, 16, _LANE))
    rep = rep.reshape(_RB, _LANE)
    rr = lax.broadcasted_iota(jnp.int32, (_RB, _LANE), 0)
    cc = lax.broadcasted_iota(jnp.int32, (_RB, _LANE), 1)
    lmask = (cc // _FOLD == rr % 16).astype(jnp.float32)
    x1 = rep * lmask
    rc = lax.broadcasted_iota(jnp.int32, (_LANE, _LANE), 0)
    rl = lax.broadcasted_iota(jnp.int32, (_LANE, _LANE), 1)
    rmat = (rl // _D == rc % _FOLD).astype(jnp.float32)
    return jnp.dot(x1, rmat, preferred_element_type=jnp.float32,
                   precision=lax.Precision.HIGHEST)


def _tc1_body(x_ref, w1_ref, p0_ref, p1_ref, g_ref, dinv_ref):
    wide = _expand_dinv(p0_ref[0], p1_ref[0])
    h = jnp.dot(x_ref[...], w1_ref[...], preferred_element_type=jnp.float32)
    g_ref[...] = h * wide
    dinv_ref[...] = wide


def _tc2_body(a0_ref, a1_ref, g1_ref, dinv_ref, b1_ref, w2_ref, g2_ref):
    dinv = dinv_ref[...]
    z = (a0_ref[0] + a1_ref[0] + g1_ref[...]) * dinv + b1_ref[...]
    z = jnp.maximum(z, 0.0)
    g2_ref[...] = jnp.dot(z, w2_ref[...],
                          preferred_element_type=jnp.float32) * dinv


_RB3 = 256
_GRID3 = (49,)


def _tc3_body(a0_ref, a1_ref, g2_ref, dinv_ref, b2_ref, k1_ref, cb1_ref,
              k2_ref, cb2_ref, o1_ref, o2_ref):
    z = (a0_ref[0] + a1_ref[0] + g2_ref[...]) * dinv_ref[...] + b2_ref[...]
    z = jnp.maximum(z, 0.0)
    y = jnp.broadcast_to(z[:, None, :], (_RB3, _FOLD, _LANE))
    y = y.reshape(_RB3 * _FOLD, _LANE)
    n_i = lax.broadcasted_iota(jnp.int32, (_RB3 * _FOLD, _LANE), 0)
    l_i = lax.broadcasted_iota(jnp.int32, (_RB3 * _FOLD, _LANE), 1)
    ycom = y * (l_i // _D == n_i % _FOLD).astype(jnp.float32)
    o1_ref[...] = jnp.dot(ycom, k1_ref[...],
                          preferred_element_type=jnp.float32) + cb1_ref[...]
    o2_ref[...] = jnp.dot(ycom, k2_ref[...],
                          preferred_element_type=jnp.float32) + cb2_ref[...]


def _row_spec(w):
    return pl.BlockSpec((_RB, w), lambda i: (i, 0))


def _full_spec(h, w):
    return pl.BlockSpec((h, w), lambda i: (0, 0))


def _part_spec(p, h, w):
    return pl.BlockSpec((1, h, w), lambda i, _p=p: (_p, i, 0))


_tc1 = pl.pallas_call(
    _tc1_body,
    grid=_GRID,
    in_specs=[_row_spec(48), _full_spec(48, _LANE),
              _part_spec(0, _DB, _LANE), _part_spec(1, _DB, _LANE)],
    out_specs=[_row_spec(_LANE), _row_spec(_LANE)],
    out_shape=[
        jax.ShapeDtypeStruct((_NF, _LANE), jnp.float32),
        jax.ShapeDtypeStruct((_NF, _LANE), jnp.float32),
    ],
)

_tc2 = pl.pallas_call(
    _tc2_body,
    grid=_GRID,
    in_specs=[_part_spec(0, _RB, _LANE), _part_spec(1, _RB, _LANE),
              _row_spec(_LANE), _row_spec(_LANE),
              _full_spec(1, _LANE), _full_spec(_LANE, _LANE)],
    out_specs=[_row_spec(_LANE)],
    out_shape=[jax.ShapeDtypeStruct((_NF, _LANE), jnp.float32)],
)

_tc3 = pl.pallas_call(
    _tc3_body,
    grid=_GRID3,
    in_specs=[_part_spec(0, _RB3, _LANE), _part_spec(1, _RB3, _LANE),
              pl.BlockSpec((_RB3, _LANE), lambda i: (i, 0)),
              pl.BlockSpec((_RB3, _LANE), lambda i: (i, 0)),
              _full_spec(1, _LANE), _full_spec(_LANE, 13),
              _full_spec(1, 13), _full_spec(_LANE, 8),
              _full_spec(1, 8)],
    out_specs=[pl.BlockSpec((_RB3 * _FOLD, 13), lambda i: (i, 0)),
               pl.BlockSpec((_RB3 * _FOLD, 8), lambda i: (i, 0))],
    out_shape=[
        jax.ShapeDtypeStruct((_N, 13), jnp.float32),
        jax.ShapeDtypeStruct((_N, 8), jnp.float32),
    ],
)

_EYE8 = np.eye(_FOLD, dtype=np.float32)


def kernel(x, edge_index, W1, b1, W2, b2, CW1, Cb1, CW2, Cb2):
    pad_i = jnp.arange(_EPAD - _E, dtype=jnp.int32)
    dst2d = jnp.concatenate([edge_index[1],
                             _N + pad_i % _PADR]).reshape(_BLKS, _LANE)
    degp = _deg(dst2d)
    src2d = jnp.concatenate([edge_index[0],
                             pad_i % _N]).reshape(_BLKS, _LANE)

    xr = x.reshape(_N // _FOLD, _FOLD * 6)
    xf = jnp.concatenate(
        [xr, jnp.zeros((_NF - _N // _FOLD, _FOLD * 6), jnp.float32)])

    w1big = jnp.kron(_EYE8, W1)
    w2big = jnp.kron(_EYE8, W2)
    k1 = jnp.tile(CW1, (_FOLD, 1))
    k2 = jnp.tile(CW2, (_FOLD, 1))
    b1t = jnp.tile(b1, _FOLD).reshape(1, _LANE)
    b2t = jnp.tile(b2, _FOLD).reshape(1, _LANE)
    cb1s = Cb1.reshape(1, 13)
    cb2s = Cb2.reshape(1, 8)

    dg = degp.reshape(_NC, _NDEG, _LANE)

    g1f, dinvw = _tc1(xf, w1big, dg, dg)
    acc1 = _agg(g1f.reshape(_NT, _D), src2d, dst2d)
    (g2f,) = _tc2(acc1.reshape(_NC, _NF, _LANE),
                  acc1.reshape(_NC, _NF, _LANE),
                  g1f, dinvw, b1t, w2big)
    acc2 = _agg(g2f.reshape(_NT, _D), src2d, dst2d)
    out1, out2 = _tc3(acc2.reshape(_NC, _NF, _LANE),
                      acc2.reshape(_NC, _NF, _LANE),
                      g2f, dinvw, b2t, k1, cb1s, k2, cb2s)
    return (out1, out2)
